# baseline (speedup 1.0000x reference)
import functools

import jax
import jax.numpy as jnp
from jax import lax
from jax.experimental import pallas as pl
from jax.experimental.pallas import tpu as pltpu
from jax.experimental.pallas import tpu_sc as plsc

N_USERS_K = 5000
N_ITEMS_K = 5000
N_TOT = N_USERS_K + N_ITEMS_K
D = 128
E_EDGES = 320000

NS = 16
CHUNK = 128
CPT = 160
HPT = CPT // 2
NCH = NS * CPT
E_PAD = NCH * CHUNK
ROWS_PER_TILE = 624


def _sc_round_body(emb_hbm, spk_hbm, dst_hbm, w_hbm, out_hbm,
                   acc, slab, i0, i1, i2, i3, d0, d1, w0, w1, rows0, rows1,
                   gsem0, gsem1, ssem0, ssem1,
                   dsem0, dsem1, wsem0, wsem1):
    cid = lax.axis_index("c")
    sid = lax.axis_index("s")
    rows = (rows0, rows1)
    ib = (i0, i1, i2, i3)
    db = (d0, d1)
    wb = (w0, w1)
    gsem = (gsem0, gsem1)
    ssem = (ssem0, ssem1)
    dsem = (dsem0, dsem1)
    wsem = (wsem0, wsem1)
    base_row = sid * ROWS_PER_TILE
    tail_base = NS * ROWS_PER_TILE
    tail_rows = N_TOT - tail_base
    cbase = sid * CPT

    def start_dst(j, b):
        pltpu.async_copy(dst_hbm.at[cbase + j], db[b].at[0], dsem[b])

    def wait_dst(b):
        pltpu.make_async_copy(dst_hbm.at[0], db[b].at[0], dsem[b]).wait()

    def start_w(j, b):
        pltpu.async_copy(w_hbm.at[cbase + j], wb[b].at[0], wsem[b])

    def wait_w(b):
        pltpu.make_async_copy(w_hbm.at[0], wb[b].at[0], wsem[b]).wait()

    def build_idx(row, half, i):
        for g in range(4):
            v = slab[row, pl.ds(64 * half + g * 16, 16)]
            ib[i][0, pl.ds(g * 16, 16)] = v & 0xFFFF
            ib[i][0, pl.ds(64 + g * 16, 16)] = v >> 16

    def start_gather(b, i):
        pltpu.async_copy(emb_hbm.at[ib[i].at[0]], rows[b], gsem[b])

    def wait_gather(b):
        pltpu.make_async_copy(emb_hbm.at[i0.at[0]], rows[b], gsem[b]).wait()

    def start_scatter(b):
        pltpu.async_copy(rows[b], acc.at[db[b].at[0]], ssem[b], add=True)

    def wait_scatter(b):
        pltpu.make_async_copy(rows[b], acc.at[d0.at[0]], ssem[b]).wait()

    def scale(b):
        rv = rows[b]
        wrow = wb[b]

        def scale_group(g, c2):
            wvec = wrow[0, pl.ds(g * 16, 16)]
            for k in range(16):
                we = wvec[k]
                row = g * 16 + k
                for l in range(D // 16):
                    rv[row, pl.ds(l * 16, 16)] = (
                        rv[row, pl.ds(l * 16, 16)] * we)
            return c2
        lax.fori_loop(0, CHUNK // 16, scale_group, 0)

    @pl.when(cid == 0)
    def _zero_phase():
        def zero_rows(r, carry):
            for l in range(D // 16):
                rows0[r, pl.ds(l * 16, 16)] = jnp.zeros((16,), jnp.float32)
            return carry
        lax.fori_loop(0, CHUNK, zero_rows, 0)
        for k in range(-(-ROWS_PER_TILE // CHUNK)):
            nr = min(CHUNK, ROWS_PER_TILE - k * CHUNK)
            pltpu.sync_copy(rows0.at[pl.ds(0, nr)],
                            acc.at[pl.ds(base_row + k * CHUNK, nr)])

        @pl.when(sid == 0)
        def _zero_tail():
            pltpu.sync_copy(rows0.at[pl.ds(0, tail_rows)],
                            acc.at[pl.ds(tail_base, tail_rows)])
    plsc.subcore_barrier()

    def _edge_phase():
        pltpu.sync_copy(spk_hbm.at[pl.ds(sid * HPT, HPT)], slab)

        start_dst(0, 0)
        start_w(0, 0)
        build_idx(0, 0, 0)
        start_gather(0, 0)
        build_idx(0, 1, 1)
        wait_gather(0)
        start_gather(1, 1)
        start_dst(1, 1)
        start_w(1, 1)
        build_idx(1, 0, 2)
        wait_w(0)
        scale(0)
        wait_dst(0)
        start_scatter(0)

        def step(j, row2, b, inext, ibuild, hbuild):
            wait_gather(b)
            wait_scatter(1 - b)
            start_gather(1 - b, inext)
            start_dst(j + 1, 1 - b)
            start_w(j + 1, 1 - b)
            build_idx(row2, hbuild, ibuild)
            wait_w(b)
            scale(b)
            wait_dst(b)
            start_scatter(b)

        def body(t, carry):
            j = 4 * t + 1
            step(j, 2 * t + 1, 1, 2, 3, 1)
            step(j + 1, 2 * t + 2, 0, 3, 0, 0)
            step(j + 2, 2 * t + 2, 1, 0, 1, 1)
            step(j + 3, 2 * t + 3, 0, 1, 2, 0)
            return carry
        lax.fori_loop(0, (CPT - 4) // 4, body, 0)

        wait_gather(1)
        wait_scatter(0)
        start_gather(0, (CPT - 2) % 4)
        start_dst(CPT - 2, 0)
        start_w(CPT - 2, 0)
        build_idx(HPT - 1, 1, (CPT - 1) % 4)
        wait_w(1)
        scale(1)
        wait_dst(1)
        start_scatter(1)

        wait_gather(0)
        wait_scatter(1)
        start_gather(1, (CPT - 1) % 4)
        start_dst(CPT - 1, 1)
        start_w(CPT - 1, 1)
        wait_w(0)
        scale(0)
        wait_dst(0)
        start_scatter(0)

        wait_gather(1)
        wait_scatter(0)
        wait_w(1)
        scale(1)
        wait_dst(1)
        start_scatter(1)
        wait_scatter(1)
    _edge_phase()
    plsc.subcore_barrier()

    @pl.when(cid == 0)
    def _readout():
        pltpu.sync_copy(acc.at[pl.ds(base_row, ROWS_PER_TILE)],
                        out_hbm.at[pl.ds(base_row, ROWS_PER_TILE)])

        @pl.when(sid == 0)
        def _write_tail():
            pltpu.sync_copy(acc.at[pl.ds(tail_base, tail_rows)],
                            out_hbm.at[pl.ds(tail_base, tail_rows)])


@jax.jit
def _sc_round(emb, spk, dst2d, w2d):
    mesh = plsc.VectorSubcoreMesh(core_axis_name="c", subcore_axis_name="s")
    return pl.kernel(
        _sc_round_body,
        out_type=jax.ShapeDtypeStruct((N_TOT, D), jnp.float32),
        mesh=mesh,
        scratch_types=[
            pltpu.VMEM_SHARED((N_TOT, D), jnp.float32),
            pltpu.VMEM((HPT, CHUNK), jnp.int32),
            pltpu.VMEM((1, CHUNK), jnp.int32),
            pltpu.VMEM((1, CHUNK), jnp.int32),
            pltpu.VMEM((1, CHUNK), jnp.int32),
            pltpu.VMEM((1, CHUNK), jnp.int32),
            pltpu.VMEM((1, CHUNK), jnp.int32),
            pltpu.VMEM((1, CHUNK), jnp.int32),
            pltpu.VMEM((1, CHUNK), jnp.float32),
            pltpu.VMEM((1, CHUNK), jnp.float32),
            pltpu.VMEM((CHUNK, D), jnp.float32),
            pltpu.VMEM((CHUNK, D), jnp.float32),
            pltpu.SemaphoreType.DMA,
            pltpu.SemaphoreType.DMA,
            pltpu.SemaphoreType.DMA,
            pltpu.SemaphoreType.DMA,
            pltpu.SemaphoreType.DMA,
            pltpu.SemaphoreType.DMA,
            pltpu.SemaphoreType.DMA,
            pltpu.SemaphoreType.DMA,
        ],
    )(emb, spk, dst2d, w2d)


def _final_body(e0_ref, e1_ref, e2_ref, o_ref):
    o_ref[...] = (e0_ref[...] + e1_ref[...] + e2_ref[...]) * (1.0 / 3.0)


_TC_BLK = 1000


@jax.jit
def _final(emb0, emb1, emb2):
    spec = pl.BlockSpec((_TC_BLK, D), lambda i: (i, 0))
    return pl.pallas_call(
        _final_body,
        grid=(N_TOT // _TC_BLK,),
        in_specs=[spec] * 3,
        out_specs=spec,
        out_shape=jax.ShapeDtypeStruct((N_TOT, D), jnp.float32),
    )(emb0, emb1, emb2)


def kernel(edge_index, edge_weight, user_emb, item_emb):
    emb0 = jnp.concatenate([user_emb, item_emb], axis=0)
    dst = edge_index[0]
    src = edge_index[1]
    pad = E_PAD - E_EDGES
    src2d = jnp.pad(src, (0, pad)).reshape(NCH, CHUNK)
    dst2d = jnp.pad(dst, (0, pad)).reshape(NCH, CHUNK)
    w2d = jnp.pad(edge_weight, (0, pad)).reshape(NCH, CHUNK)
    spk = (src2d[:, :64] | (src2d[:, 64:] << 16)).reshape(NCH // 2, CHUNK)

    emb1 = _sc_round(emb0, spk, dst2d, w2d)
    emb2 = _sc_round(emb1, spk, dst2d, w2d)
    out = _final(emb0, emb1, emb2)
    return (out[:N_USERS_K], out[N_USERS_K:])


# speedup vs baseline: 1.7952x; 1.7952x over previous
import functools

import jax
import jax.numpy as jnp
from jax import lax
from jax.experimental import pallas as pl
from jax.experimental.pallas import tpu as pltpu
from jax.experimental.pallas import tpu_sc as plsc

N_USERS_K = 5000
N_ITEMS_K = 5000
N_TOT = N_USERS_K + N_ITEMS_K
D = 128
E_EDGES = 320000

NC = 2
NS = 16
NW = NC * NS
CHUNK = 128
CPW0 = 128
CPW1 = 32
NCH = NS * (CPW0 + CPW1)
E_PAD = NCH * CHUNK
ROWS_PER_TILE = 624


def _sc_round_body(emb_hbm, src_hbm, dst_hbm, w_hbm, out_hbm,
                   acc, src_sl, dst0, dst1, w0, w1, rows0, rows1,
                   gsem0, gsem1, ssem0, ssem1, dsem0, dsem1,
                   wsem0, wsem1):
    cid = lax.axis_index("c")
    sid = lax.axis_index("s")
    rows = (rows0, rows1)
    dstb = (dst0, dst1)
    wb = (w0, w1)
    gsem = (gsem0, gsem1)
    ssem = (ssem0, ssem1)
    dsem = (dsem0, dsem1)
    wsem = (wsem0, wsem1)

    def zero_rows(r, carry):
        for l in range(D // 16):
            rows0[r, pl.ds(l * 16, 16)] = jnp.zeros((16,), jnp.float32)
        return carry
    lax.fori_loop(0, CHUNK, zero_rows, 0)
    base_row = sid * ROWS_PER_TILE
    for k in range(-(-ROWS_PER_TILE // CHUNK)):
        nr = min(CHUNK, ROWS_PER_TILE - k * CHUNK)
        pltpu.sync_copy(rows0.at[pl.ds(0, nr)],
                        acc.at[pl.ds(base_row + k * CHUNK, nr)])
    tail_base = NS * ROWS_PER_TILE
    tail_rows = N_TOT - tail_base

    @pl.when(sid == 0)
    def _zero_tail():
        pltpu.sync_copy(rows0.at[pl.ds(0, tail_rows)],
                        acc.at[pl.ds(tail_base, tail_rows)])
    plsc.subcore_barrier()

    def start_gather(j, b):
        pltpu.async_copy(emb_hbm.at[src_sl.at[j]], rows[b], gsem[b])

    def wait_gather(b):
        pltpu.make_async_copy(emb_hbm.at[src_sl.at[0]], rows[b],
                              gsem[b]).wait()

    def start_dst(r, b):
        pltpu.async_copy(dst_hbm.at[r], dstb[b].at[0], dsem[b])

    def wait_dst(b):
        pltpu.make_async_copy(dst_hbm.at[0], dstb[b].at[0],
                              dsem[b]).wait()

    def start_w(r, b):
        pltpu.async_copy(w_hbm.at[r], wb[b].at[0], wsem[b])

    def wait_w(b):
        pltpu.make_async_copy(w_hbm.at[0], wb[b].at[0], wsem[b]).wait()

    def start_scatter(b):
        pltpu.async_copy(rows[b], acc.at[dstb[b].at[0]], ssem[b], add=True)

    def wait_scatter(b):
        pltpu.make_async_copy(rows[b], acc.at[dstb[b].at[0]],
                              ssem[b]).wait()

    def scale(b):
        rv = rows[b]
        wrow = wb[b]

        def scale_group(g, c2):
            wvec = wrow[0, pl.ds(g * 16, 16)]
            for e in range(16):
                we = wvec[e]
                row = g * 16 + e
                for l in range(D // 16):
                    rv[row, pl.ds(l * 16, 16)] = (
                        rv[row, pl.ds(l * 16, 16)] * we)
            return c2
        lax.fori_loop(0, CHUNK // 16, scale_group, 0)

    def run_core(cpw, start):
        pltpu.sync_copy(src_hbm.at[pl.ds(start, cpw)],
                        src_sl.at[pl.ds(0, cpw)])

        start_dst(start + 0, 0)
        start_w(start + 0, 0)
        start_gather(0, 0)
        wait_gather(0)
        start_dst(start + 1, 1)
        start_w(start + 1, 1)
        start_gather(1, 1)
        wait_w(0)
        scale(0)
        wait_dst(0)
        start_scatter(0)

        def step(j, b):
            wait_gather(b)
            wait_scatter(1 - b)
            start_dst(start + j + 1, 1 - b)
            start_w(start + j + 1, 1 - b)
            start_gather(j + 1, 1 - b)
            wait_w(b)
            scale(b)
            wait_dst(b)
            start_scatter(b)

        def body(t, carry):
            step(2 * t + 1, 1)
            step(2 * t + 2, 0)
            return carry
        lax.fori_loop(0, (cpw - 2) // 2, body, 0)

        wait_gather(1)
        wait_w(1)
        scale(1)
        wait_dst(1)
        start_scatter(1)
        wait_scatter(0)
        wait_scatter(1)

    @pl.when(cid == 0)
    def _core0():
        run_core(CPW0, sid * CPW0)

    @pl.when(cid == 1)
    def _core1():
        run_core(CPW1, NS * CPW0 + sid * CPW1)
    plsc.subcore_barrier()

    pltpu.sync_copy(acc.at[pl.ds(base_row, ROWS_PER_TILE)],
                    out_hbm.at[pl.ds(cid * N_TOT + base_row, ROWS_PER_TILE)])

    @pl.when(sid == 0)
    def _write_tail():
        pltpu.sync_copy(acc.at[pl.ds(tail_base, tail_rows)],
                        out_hbm.at[pl.ds(cid * N_TOT + tail_base, tail_rows)])


@jax.jit
def _sc_round(emb, src2d, dst2d, w2d):
    mesh = plsc.VectorSubcoreMesh(core_axis_name="c", subcore_axis_name="s")
    return pl.kernel(
        _sc_round_body,
        out_type=jax.ShapeDtypeStruct((NC * N_TOT, D), jnp.float32),
        mesh=mesh,
        scratch_types=[
            pltpu.VMEM_SHARED((N_TOT, D), jnp.float32),
            pltpu.VMEM((CPW0, CHUNK), jnp.int32),
            pltpu.VMEM((1, CHUNK), jnp.int32),
            pltpu.VMEM((1, CHUNK), jnp.int32),
            pltpu.VMEM((1, CHUNK), jnp.float32),
            pltpu.VMEM((1, CHUNK), jnp.float32),
            pltpu.VMEM((CHUNK, D), jnp.float32),
            pltpu.VMEM((CHUNK, D), jnp.float32),
            pltpu.SemaphoreType.DMA,
            pltpu.SemaphoreType.DMA,
            pltpu.SemaphoreType.DMA,
            pltpu.SemaphoreType.DMA,
            pltpu.SemaphoreType.DMA,
            pltpu.SemaphoreType.DMA,
            pltpu.SemaphoreType.DMA,
            pltpu.SemaphoreType.DMA,
        ],
    )(emb, src2d, dst2d, w2d)


def _add2_body(a_ref, b_ref, o_ref):
    o_ref[...] = a_ref[...] + b_ref[...]


def _final_body(e0_ref, e1_ref, p0_ref, p1_ref, o_ref):
    o_ref[...] = (e0_ref[...] + e1_ref[...] + p0_ref[...] + p1_ref[...]) * (1.0 / 3.0)


_TC_BLK = 1000


def _tc_specs(n_in):
    spec = pl.BlockSpec((_TC_BLK, D), lambda i: (i, 0))
    return dict(
        grid=(N_TOT // _TC_BLK,),
        in_specs=[spec] * n_in,
        out_specs=spec,
        out_shape=jax.ShapeDtypeStruct((N_TOT, D), jnp.float32),
    )


@jax.jit
def _combine2(p):
    return pl.pallas_call(_add2_body, **_tc_specs(2))(p[:N_TOT], p[N_TOT:])


@jax.jit
def _final(emb0, emb1, p2):
    return pl.pallas_call(_final_body, **_tc_specs(4))(
        emb0, emb1, p2[:N_TOT], p2[N_TOT:])


def kernel(edge_index, edge_weight, user_emb, item_emb):
    emb0 = jnp.concatenate([user_emb, item_emb], axis=0)
    dst = edge_index[0]
    src = edge_index[1]
    pad = E_PAD - E_EDGES
    src2d = jnp.pad(src, (0, pad)).reshape(NCH, CHUNK)
    dst2d = jnp.pad(dst, (0, pad)).reshape(NCH, CHUNK)
    w2d = jnp.pad(edge_weight, (0, pad)).reshape(NCH, CHUNK)

    p1 = _sc_round(emb0, src2d, dst2d, w2d)
    emb1 = _combine2(p1)
    p2 = _sc_round(emb1, src2d, dst2d, w2d)
    out = _final(emb0, emb1, p2)
    return (out[:N_USERS_K], out[N_USERS_K:])


# speedup vs baseline: 1.8698x; 1.0415x over previous
import functools

import jax
import jax.numpy as jnp
from jax import lax
from jax.experimental import pallas as pl
from jax.experimental.pallas import tpu as pltpu
from jax.experimental.pallas import tpu_sc as plsc

N_USERS_K = 5000
N_ITEMS_K = 5000
N_TOT = N_USERS_K + N_ITEMS_K
D = 128
E_EDGES = 320000

NC = 2
NS = 16
NW = NC * NS
CHUNK = 128
CPW0 = 136
CPW1 = 24
NCH = NS * (CPW0 + CPW1)
E_PAD = NCH * CHUNK
ROWS_PER_TILE = 624


def _sc_round_body(emb_hbm, src_hbm, dst_hbm, w_hbm, out_hbm,
                   acc, src_sl, dst0, dst1, w0, w1, rows0, rows1,
                   gsem0, gsem1, ssem0, ssem1, dsem0, dsem1,
                   wsem0, wsem1):
    cid = lax.axis_index("c")
    sid = lax.axis_index("s")
    rows = (rows0, rows1)
    dstb = (dst0, dst1)
    wb = (w0, w1)
    gsem = (gsem0, gsem1)
    ssem = (ssem0, ssem1)
    dsem = (dsem0, dsem1)
    wsem = (wsem0, wsem1)

    def zero_rows(r, carry):
        for l in range(D // 16):
            rows0[r, pl.ds(l * 16, 16)] = jnp.zeros((16,), jnp.float32)
        return carry
    lax.fori_loop(0, CHUNK, zero_rows, 0)
    base_row = sid * ROWS_PER_TILE
    for k in range(-(-ROWS_PER_TILE // CHUNK)):
        nr = min(CHUNK, ROWS_PER_TILE - k * CHUNK)
        pltpu.sync_copy(rows0.at[pl.ds(0, nr)],
                        acc.at[pl.ds(base_row + k * CHUNK, nr)])
    tail_base = NS * ROWS_PER_TILE
    tail_rows = N_TOT - tail_base

    @pl.when(sid == 0)
    def _zero_tail():
        pltpu.sync_copy(rows0.at[pl.ds(0, tail_rows)],
                        acc.at[pl.ds(tail_base, tail_rows)])
    plsc.subcore_barrier()

    def start_gather(j, b):
        pltpu.async_copy(emb_hbm.at[src_sl.at[j]], rows[b], gsem[b])

    def wait_gather(b):
        pltpu.make_async_copy(emb_hbm.at[src_sl.at[0]], rows[b],
                              gsem[b]).wait()

    def start_dst(r, b):
        pltpu.async_copy(dst_hbm.at[r], dstb[b].at[0], dsem[b])

    def wait_dst(b):
        pltpu.make_async_copy(dst_hbm.at[0], dstb[b].at[0],
                              dsem[b]).wait()

    def start_w(r, b):
        pltpu.async_copy(w_hbm.at[r], wb[b].at[0], wsem[b])

    def wait_w(b):
        pltpu.make_async_copy(w_hbm.at[0], wb[b].at[0], wsem[b]).wait()

    def start_scatter(b):
        pltpu.async_copy(rows[b], acc.at[dstb[b].at[0]], ssem[b], add=True)

    def wait_scatter(b):
        pltpu.make_async_copy(rows[b], acc.at[dstb[b].at[0]],
                              ssem[b]).wait()

    def scale(b):
        rv = rows[b]
        wrow = wb[b]

        def scale_group(g, c2):
            wvec = wrow[0, pl.ds(g * 16, 16)]
            for e in range(16):
                we = wvec[e]
                row = g * 16 + e
                for l in range(D // 16):
                    rv[row, pl.ds(l * 16, 16)] = (
                        rv[row, pl.ds(l * 16, 16)] * we)
            return c2
        lax.fori_loop(0, CHUNK // 16, scale_group, 0)

    def run_core(cpw, start):
        pltpu.sync_copy(src_hbm.at[pl.ds(start, cpw)],
                        src_sl.at[pl.ds(0, cpw)])

        start_dst(start + 0, 0)
        start_w(start + 0, 0)
        start_gather(0, 0)
        wait_gather(0)
        start_dst(start + 1, 1)
        start_w(start + 1, 1)
        start_gather(1, 1)
        wait_w(0)
        scale(0)
        wait_dst(0)
        start_scatter(0)

        def step(j, b):
            wait_gather(b)
            wait_scatter(1 - b)
            start_dst(start + j + 1, 1 - b)
            start_w(start + j + 1, 1 - b)
            start_gather(j + 1, 1 - b)
            wait_w(b)
            scale(b)
            wait_dst(b)
            start_scatter(b)

        def body(t, carry):
            step(2 * t + 1, 1)
            step(2 * t + 2, 0)
            return carry
        lax.fori_loop(0, (cpw - 2) // 2, body, 0)

        wait_gather(1)
        wait_w(1)
        scale(1)
        wait_dst(1)
        start_scatter(1)
        wait_scatter(0)
        wait_scatter(1)

    @pl.when(cid == 0)
    def _core0():
        run_core(CPW0, sid * CPW0)

    @pl.when(cid == 1)
    def _core1():
        run_core(CPW1, NS * CPW0 + sid * CPW1)
    plsc.subcore_barrier()

    pltpu.sync_copy(acc.at[pl.ds(base_row, ROWS_PER_TILE)],
                    out_hbm.at[pl.ds(cid * N_TOT + base_row, ROWS_PER_TILE)])

    @pl.when(sid == 0)
    def _write_tail():
        pltpu.sync_copy(acc.at[pl.ds(tail_base, tail_rows)],
                        out_hbm.at[pl.ds(cid * N_TOT + tail_base, tail_rows)])


@jax.jit
def _sc_round(emb, src2d, dst2d, w2d):
    mesh = plsc.VectorSubcoreMesh(core_axis_name="c", subcore_axis_name="s")
    return pl.kernel(
        _sc_round_body,
        out_type=jax.ShapeDtypeStruct((NC * N_TOT, D), jnp.float32),
        mesh=mesh,
        scratch_types=[
            pltpu.VMEM_SHARED((N_TOT, D), jnp.float32),
            pltpu.VMEM((CPW0, CHUNK), jnp.int32),
            pltpu.VMEM((1, CHUNK), jnp.int32),
            pltpu.VMEM((1, CHUNK), jnp.int32),
            pltpu.VMEM((1, CHUNK), jnp.float32),
            pltpu.VMEM((1, CHUNK), jnp.float32),
            pltpu.VMEM((CHUNK, D), jnp.float32),
            pltpu.VMEM((CHUNK, D), jnp.float32),
            pltpu.SemaphoreType.DMA,
            pltpu.SemaphoreType.DMA,
            pltpu.SemaphoreType.DMA,
            pltpu.SemaphoreType.DMA,
            pltpu.SemaphoreType.DMA,
            pltpu.SemaphoreType.DMA,
            pltpu.SemaphoreType.DMA,
            pltpu.SemaphoreType.DMA,
        ],
    )(emb, src2d, dst2d, w2d)


def _add2_body(a_ref, b_ref, o_ref):
    o_ref[...] = a_ref[...] + b_ref[...]


def _final_body(e0_ref, e1_ref, p0_ref, p1_ref, o_ref):
    o_ref[...] = (e0_ref[...] + e1_ref[...] + p0_ref[...] + p1_ref[...]) * (1.0 / 3.0)


_TC_BLK = 1000


def _tc_specs(n_in):
    spec = pl.BlockSpec((_TC_BLK, D), lambda i: (i, 0))
    return dict(
        grid=(N_TOT // _TC_BLK,),
        in_specs=[spec] * n_in,
        out_specs=spec,
        out_shape=jax.ShapeDtypeStruct((N_TOT, D), jnp.float32),
    )


@jax.jit
def _combine2(p):
    return pl.pallas_call(_add2_body, **_tc_specs(2))(p[:N_TOT], p[N_TOT:])


@jax.jit
def _final(emb0, emb1, p2):
    return pl.pallas_call(_final_body, **_tc_specs(4))(
        emb0, emb1, p2[:N_TOT], p2[N_TOT:])


def kernel(edge_index, edge_weight, user_emb, item_emb):
    emb0 = jnp.concatenate([user_emb, item_emb], axis=0)
    dst = edge_index[0]
    src = edge_index[1]
    pad = E_PAD - E_EDGES
    src2d = jnp.pad(src, (0, pad)).reshape(NCH, CHUNK)
    dst2d = jnp.pad(dst, (0, pad)).reshape(NCH, CHUNK)
    w2d = jnp.pad(edge_weight, (0, pad)).reshape(NCH, CHUNK)

    p1 = _sc_round(emb0, src2d, dst2d, w2d)
    emb1 = _combine2(p1)
    p2 = _sc_round(emb1, src2d, dst2d, w2d)
    out = _final(emb0, emb1, p2)
    return (out[:N_USERS_K], out[N_USERS_K:])
